# baseline (speedup 1.0000x reference)
import functools

import jax
import jax.numpy as jnp
from jax import lax
from jax.experimental import pallas as pl
from jax.experimental.pallas import tpu as pltpu
from jax.experimental.pallas import tpu_sc as plsc

N_NODES = 10000
FDIM = 128
BLK = 1000
GRID = N_NODES // BLK

NCORES = 2
NSUB = 16
NWORK = NCORES * NSUB
CH = 64
CPW = 160
TOT_CHUNKS = NWORK * CPW
EPAD = TOT_CHUNKS * CH
NPAD = 10240
RPT = NPAD // NSUB
GC = 8
C0 = 256
C1 = 64

_mesh = plsc.VectorSubcoreMesh(core_axis_name="c", subcore_axis_name="s")


def _seg_rows_body(p_hbm, src_hbm, dst_hbm, out_hbm,
                   src_v, dst_v, rows_a, rows_b, rows_c, rows_d, zbuf, acc_sh,
                   sem_a, sem_b, sem_c, sem_d):
    c = lax.axis_index("c")
    s = lax.axis_index("s")
    cnt = C0 + c * (C1 - C0)
    start = c * (NSUB * C0) + s * cnt
    ngroups = (C0 // GC) + c * ((C1 // GC) - (C0 // GC))

    zero16 = jnp.zeros((16,), jnp.float32)

    def _zrow(i, _):
        for j in range(FDIM // 16):
            zbuf[i, pl.ds(j * 16, 16)] = zero16
        return 0
    lax.fori_loop(0, 16, _zrow, 0)

    def _zacc(k, _):
        pltpu.sync_copy(zbuf, acc_sh.at[pl.ds(s * RPT + k * 16, 16)])
        return 0
    lax.fori_loop(0, RPT // 16, _zacc, 0)

    plsc.subcore_barrier()

    def _group(g, _):
        pltpu.sync_copy(src_hbm.at[pl.ds(start + g * GC, GC)], src_v)
        pltpu.sync_copy(dst_hbm.at[pl.ds(start + g * GC, GC)], dst_v)
        copies = [None, None, None, None]
        bufs = [rows_a, rows_b, rows_c, rows_d]
        sems = [sem_a, sem_b, sem_c, sem_d]
        for k in range(3):
            copies[k] = pltpu.async_copy(p_hbm.at[src_v.at[k]], bufs[k], sems[k])
        for j in range(GC):
            if j + 3 < GC:
                copies[(j + 3) % 4] = pltpu.async_copy(
                    p_hbm.at[src_v.at[j + 3]], bufs[(j + 3) % 4], sems[(j + 3) % 4])
            copies[j % 4].wait()
            pltpu.sync_copy(bufs[j % 4], acc_sh.at[dst_v.at[j]], add=True)
        return 0
    lax.fori_loop(0, ngroups, _group, 0)

    plsc.subcore_barrier()

    pltpu.sync_copy(acc_sh.at[pl.ds(s * RPT, RPT)],
                    out_hbm.at[c, pl.ds(s * RPT, RPT)])


_seg_rows = functools.partial(
    pl.kernel, _seg_rows_body, mesh=_mesh,
    out_type=jax.ShapeDtypeStruct((NCORES, NPAD, FDIM), jnp.float32),
    scratch_types=[
        pltpu.VMEM((GC, CH), jnp.int32),
        pltpu.VMEM((GC, CH), jnp.int32),
        pltpu.VMEM((CH, FDIM), jnp.float32),
        pltpu.VMEM((CH, FDIM), jnp.float32),
        pltpu.VMEM((CH, FDIM), jnp.float32),
        pltpu.VMEM((CH, FDIM), jnp.float32),
        pltpu.VMEM((16, FDIM), jnp.float32),
        pltpu.VMEM_SHARED((NPAD, FDIM), jnp.float32),
        pltpu.SemaphoreType.DMA,
        pltpu.SemaphoreType.DMA,
        pltpu.SemaphoreType.DMA,
        pltpu.SemaphoreType.DMA,
    ],
)()


def _seg_scalar_body(p_hbm, src_hbm, dst_hbm, out_hbm,
                     src_v, dst_v, vals_v, vals_w, z1d, acc_sh, sem, sem2):
    c = lax.axis_index("c")
    s = lax.axis_index("s")
    wid = c * NSUB + s

    zero16 = jnp.zeros((16,), jnp.float32)

    def _z1(i, _):
        z1d[pl.ds(i * 16, 16)] = zero16
        return 0
    lax.fori_loop(0, RPT // 16, _z1, 0)
    pltpu.sync_copy(z1d, acc_sh.at[pl.ds(s * RPT, RPT)])

    pltpu.sync_copy(src_hbm.at[pl.ds(wid * CPW, CPW)], src_v)
    pltpu.sync_copy(dst_hbm.at[pl.ds(wid * CPW, CPW)], dst_v)

    plsc.subcore_barrier()

    def _vgroup(g, _):
        base = g * GC
        copies = [None, None]
        bufs = [vals_v, vals_w]
        copies[0] = pltpu.async_copy(p_hbm.at[src_v.at[base]], vals_v, sem)
        for j in range(GC):
            if j + 1 < GC:
                copies[(j + 1) % 2] = pltpu.async_copy(
                    p_hbm.at[src_v.at[base + j + 1]], bufs[(j + 1) % 2], sem2)
            copies[j % 2].wait()
            pltpu.sync_copy(bufs[j % 2], acc_sh.at[dst_v.at[base + j]], add=True)
        return 0
    lax.fori_loop(0, CPW // GC, _vgroup, 0)

    plsc.subcore_barrier()

    pltpu.sync_copy(acc_sh.at[pl.ds(s * RPT, RPT)],
                    out_hbm.at[c, pl.ds(s * RPT, RPT)])


_seg_scalar = functools.partial(
    pl.kernel, _seg_scalar_body, mesh=_mesh,
    out_type=jax.ShapeDtypeStruct((NCORES, NPAD), jnp.float32),
    scratch_types=[
        pltpu.VMEM((CPW, CH), jnp.int32),
        pltpu.VMEM((CPW, CH), jnp.int32),
        pltpu.VMEM((CH,), jnp.float32),
        pltpu.VMEM((CH,), jnp.float32),
        pltpu.VMEM((RPT,), jnp.float32),
        pltpu.VMEM_SHARED((NPAD,), jnp.float32),
        pltpu.SemaphoreType.DMA,
        pltpu.SemaphoreType.DMA,
    ],
)()


def _layer1_body(x_ref, ws_ref, wn_ref, b_ref, s1_ref, p1_ref):
    xb = x_ref[...]
    s1_ref[...] = jnp.dot(xb, ws_ref[...], preferred_element_type=jnp.float32) + b_ref[...]
    p1_ref[...] = jnp.dot(xb, wn_ref[...], preferred_element_type=jnp.float32)


def _layer2_body(s1_ref, a0_ref, a1_ref, d0_ref, d1_ref, ws_ref, wn_ref, b_ref,
                 s2_ref, p2_ref, invd_ref):
    deg = d0_ref[...] + d1_ref[...]
    inv = 1.0 / jnp.maximum(deg, 1.0)
    h1 = jnp.maximum(s1_ref[...] + (a0_ref[...] + a1_ref[...]) * inv, 0.0)
    s2_ref[...] = jnp.dot(h1, ws_ref[...], preferred_element_type=jnp.float32) + b_ref[...]
    p2_ref[...] = jnp.dot(h1, wn_ref[...], preferred_element_type=jnp.float32)
    invd_ref[...] = inv


def _layer3_body(s2_ref, a0_ref, a1_ref, invd_ref, ws_ref, wn_ref, b_ref,
                 q3_ref, p3_ref):
    inv = invd_ref[...]
    h2 = jnp.maximum(s2_ref[...] + (a0_ref[...] + a1_ref[...]) * inv, 0.0)
    q3_ref[...] = jnp.dot(h2, ws_ref[...], preferred_element_type=jnp.float32) + b_ref[...]
    p3_ref[...] = jnp.dot(h2, wn_ref[...], preferred_element_type=jnp.float32)


def _final_body(q3_ref, a0_ref, a1_ref, invd_ref, out_ref):
    out_ref[...] = q3_ref[...] + (a0_ref[...] + a1_ref[...]) * invd_ref[...]


def _full(shape):
    return pl.BlockSpec(shape, lambda i: tuple(0 for _ in shape))


def _rows(width):
    return pl.BlockSpec((BLK, width), lambda i: (i, 0))


_layer1 = pl.pallas_call(
    _layer1_body,
    grid=(GRID,),
    in_specs=[_rows(FDIM), _full((FDIM, FDIM)), _full((FDIM, FDIM)), _full((1, FDIM))],
    out_specs=[_rows(FDIM), _rows(FDIM)],
    out_shape=[jax.ShapeDtypeStruct((N_NODES, FDIM), jnp.float32)] * 2,
)

_layer2 = pl.pallas_call(
    _layer2_body,
    grid=(GRID,),
    in_specs=[_rows(FDIM), _rows(FDIM), _rows(FDIM), _rows(1), _rows(1),
              _full((FDIM, FDIM)), _full((FDIM, FDIM)), _full((1, FDIM))],
    out_specs=[_rows(FDIM), _rows(FDIM), _rows(1)],
    out_shape=[jax.ShapeDtypeStruct((N_NODES, FDIM), jnp.float32),
               jax.ShapeDtypeStruct((N_NODES, FDIM), jnp.float32),
               jax.ShapeDtypeStruct((N_NODES, 1), jnp.float32)],
)

_layer3 = pl.pallas_call(
    _layer3_body,
    grid=(GRID,),
    in_specs=[_rows(FDIM), _rows(FDIM), _rows(FDIM), _rows(1),
              _full((FDIM, 1)), _full((FDIM, 1)), _full((1, 1))],
    out_specs=[_rows(1), _rows(1)],
    out_shape=[jax.ShapeDtypeStruct((N_NODES, 1), jnp.float32)] * 2,
)

_final = pl.pallas_call(
    _final_body,
    grid=(GRID,),
    in_specs=[_rows(1), _rows(1), _rows(1), _rows(1)],
    out_specs=_rows(1),
    out_shape=jax.ShapeDtypeStruct((N_NODES, 1), jnp.float32),
)


def kernel(x, edge_index, W1_self, W1_neigh, b1, W2_self, W2_neigh, b2,
           W3_self, W3_neigh, b3):
    src = edge_index[0].astype(jnp.int32)
    dst = edge_index[1].astype(jnp.int32)
    e = src.shape[0]
    srcp = jnp.concatenate([src, jnp.zeros((EPAD - e,), jnp.int32)]).reshape(TOT_CHUNKS, CH)
    dstp = jnp.concatenate([dst, jnp.full((EPAD - e,), NPAD - 1, jnp.int32)]).reshape(TOT_CHUNKS, CH)

    degp = _seg_scalar(jnp.ones((N_NODES,), jnp.float32), srcp, dstp)
    d0 = degp[0, :N_NODES, None]
    d1 = degp[1, :N_NODES, None]
    s1, p1 = _layer1(x, W1_self, W1_neigh, b1.reshape(1, FDIM))
    agg1 = _seg_rows(p1, srcp, dstp)
    s2, p2, invd = _layer2(s1, agg1[0, :N_NODES], agg1[1, :N_NODES], d0, d1,
                           W2_self, W2_neigh, b2.reshape(1, FDIM))
    agg2 = _seg_rows(p2, srcp, dstp)
    q3, p3 = _layer3(s2, agg2[0, :N_NODES], agg2[1, :N_NODES], invd,
                     W3_self, W3_neigh, b3.reshape(1, 1))
    agg3 = _seg_scalar(p3[:, 0], srcp, dstp)
    return _final(q3, agg3[0, :N_NODES, None], agg3[1, :N_NODES, None], invd)


# speedup vs baseline: 1.1888x; 1.1888x over previous
import functools

import jax
import jax.numpy as jnp
from jax import lax
from jax.experimental import pallas as pl
from jax.experimental.pallas import tpu as pltpu
from jax.experimental.pallas import tpu_sc as plsc

N_NODES = 10000
FDIM = 128
BLK = 1000
GRID = N_NODES // BLK

NCORES = 2
NSUB = 16
NWORK = NCORES * NSUB
CH = 128
CPW = 80
TOT_CHUNKS = NWORK * CPW
EPAD = TOT_CHUNKS * CH
NPAD = 10240
RPT = NPAD // NSUB
GC = 8
C0 = 144
C1 = 16

_mesh = plsc.VectorSubcoreMesh(core_axis_name="c", subcore_axis_name="s")


def _seg_rows_body(p_hbm, src_hbm, dst_hbm, out_hbm,
                   src_v, dst_v, rows_a, rows_b, zbuf, acc_sh, sem_a, sem_b):
    c = lax.axis_index("c")
    s = lax.axis_index("s")
    cnt = C0 + c * (C1 - C0)
    start = c * (NSUB * C0) + s * cnt
    ngroups = (C0 // GC) + c * ((C1 // GC) - (C0 // GC))

    zero16 = jnp.zeros((16,), jnp.float32)

    def _zrow(i, _):
        for j in range(FDIM // 16):
            zbuf[i, pl.ds(j * 16, 16)] = zero16
        return 0
    lax.fori_loop(0, 16, _zrow, 0)

    def _zacc(k, _):
        pltpu.sync_copy(zbuf, acc_sh.at[pl.ds(s * RPT + k * 16, 16)])
        return 0
    lax.fori_loop(0, RPT // 16, _zacc, 0)

    plsc.subcore_barrier()

    def _group(g, _):
        pltpu.sync_copy(src_hbm.at[pl.ds(start + g * GC, GC)], src_v)
        pltpu.sync_copy(dst_hbm.at[pl.ds(start + g * GC, GC)], dst_v)
        copies = [None, None]
        bufs = [rows_a, rows_b]
        copies[0] = pltpu.async_copy(p_hbm.at[src_v.at[0]], rows_a, sem_a)
        for j in range(GC):
            if j + 1 < GC:
                copies[(j + 1) % 2] = pltpu.async_copy(
                    p_hbm.at[src_v.at[j + 1]], bufs[(j + 1) % 2], sem_b)
            copies[j % 2].wait()
            pltpu.sync_copy(bufs[j % 2], acc_sh.at[dst_v.at[j]], add=True)
        return 0
    lax.fori_loop(0, ngroups, _group, 0)

    plsc.subcore_barrier()

    pltpu.sync_copy(acc_sh.at[pl.ds(s * RPT, RPT)],
                    out_hbm.at[c, pl.ds(s * RPT, RPT)])


_seg_rows = functools.partial(
    pl.kernel, _seg_rows_body, mesh=_mesh,
    out_type=jax.ShapeDtypeStruct((NCORES, NPAD, FDIM), jnp.float32),
    scratch_types=[
        pltpu.VMEM((GC, CH), jnp.int32),
        pltpu.VMEM((GC, CH), jnp.int32),
        pltpu.VMEM((CH, FDIM), jnp.float32),
        pltpu.VMEM((CH, FDIM), jnp.float32),
        pltpu.VMEM((16, FDIM), jnp.float32),
        pltpu.VMEM_SHARED((NPAD, FDIM), jnp.float32),
        pltpu.SemaphoreType.DMA,
        pltpu.SemaphoreType.DMA,
    ],
)()


def _seg_scalar_body(p_hbm, src_hbm, dst_hbm, out_hbm,
                     src_v, dst_v, vals_v, vals_w, z1d, acc_sh, sem, sem2):
    c = lax.axis_index("c")
    s = lax.axis_index("s")
    wid = c * NSUB + s

    zero16 = jnp.zeros((16,), jnp.float32)

    def _z1(i, _):
        z1d[pl.ds(i * 16, 16)] = zero16
        return 0
    lax.fori_loop(0, RPT // 16, _z1, 0)
    pltpu.sync_copy(z1d, acc_sh.at[pl.ds(s * RPT, RPT)])

    pltpu.sync_copy(src_hbm.at[pl.ds(wid * CPW, CPW)], src_v)
    pltpu.sync_copy(dst_hbm.at[pl.ds(wid * CPW, CPW)], dst_v)

    plsc.subcore_barrier()

    def _vgroup(g, _):
        base = g * GC
        copies = [None, None]
        bufs = [vals_v, vals_w]
        copies[0] = pltpu.async_copy(p_hbm.at[src_v.at[base]], vals_v, sem)
        for j in range(GC):
            if j + 1 < GC:
                copies[(j + 1) % 2] = pltpu.async_copy(
                    p_hbm.at[src_v.at[base + j + 1]], bufs[(j + 1) % 2], sem2)
            copies[j % 2].wait()
            pltpu.sync_copy(bufs[j % 2], acc_sh.at[dst_v.at[base + j]], add=True)
        return 0
    lax.fori_loop(0, CPW // GC, _vgroup, 0)

    plsc.subcore_barrier()

    pltpu.sync_copy(acc_sh.at[pl.ds(s * RPT, RPT)],
                    out_hbm.at[c, pl.ds(s * RPT, RPT)])


_seg_scalar = functools.partial(
    pl.kernel, _seg_scalar_body, mesh=_mesh,
    out_type=jax.ShapeDtypeStruct((NCORES, NPAD), jnp.float32),
    scratch_types=[
        pltpu.VMEM((CPW, CH), jnp.int32),
        pltpu.VMEM((CPW, CH), jnp.int32),
        pltpu.VMEM((CH,), jnp.float32),
        pltpu.VMEM((CH,), jnp.float32),
        pltpu.VMEM((RPT,), jnp.float32),
        pltpu.VMEM_SHARED((NPAD,), jnp.float32),
        pltpu.SemaphoreType.DMA,
        pltpu.SemaphoreType.DMA,
    ],
)()


def _layer1_body(x_ref, ws_ref, wn_ref, b_ref, s1_ref, p1_ref):
    xb = x_ref[...]
    s1_ref[...] = jnp.dot(xb, ws_ref[...], preferred_element_type=jnp.float32) + b_ref[...]
    p1_ref[...] = jnp.dot(xb, wn_ref[...], preferred_element_type=jnp.float32)


def _layer2_body(s1_ref, a0_ref, a1_ref, d0_ref, d1_ref, ws_ref, wn_ref, b_ref,
                 s2_ref, p2_ref, invd_ref):
    deg = d0_ref[...] + d1_ref[...]
    inv = 1.0 / jnp.maximum(deg, 1.0)
    h1 = jnp.maximum(s1_ref[...] + (a0_ref[...] + a1_ref[...]) * inv, 0.0)
    s2_ref[...] = jnp.dot(h1, ws_ref[...], preferred_element_type=jnp.float32) + b_ref[...]
    p2_ref[...] = jnp.dot(h1, wn_ref[...], preferred_element_type=jnp.float32)
    invd_ref[...] = inv


def _layer3_body(s2_ref, a0_ref, a1_ref, invd_ref, ws_ref, wn_ref, b_ref,
                 q3_ref, p3_ref):
    inv = invd_ref[...]
    h2 = jnp.maximum(s2_ref[...] + (a0_ref[...] + a1_ref[...]) * inv, 0.0)
    q3_ref[...] = jnp.dot(h2, ws_ref[...], preferred_element_type=jnp.float32) + b_ref[...]
    p3_ref[...] = jnp.dot(h2, wn_ref[...], preferred_element_type=jnp.float32)


def _final_body(q3_ref, a0_ref, a1_ref, invd_ref, out_ref):
    out_ref[...] = q3_ref[...] + (a0_ref[...] + a1_ref[...]) * invd_ref[...]


def _full(shape):
    return pl.BlockSpec(shape, lambda i: tuple(0 for _ in shape))


def _rows(width):
    return pl.BlockSpec((BLK, width), lambda i: (i, 0))


_layer1 = pl.pallas_call(
    _layer1_body,
    grid=(GRID,),
    in_specs=[_rows(FDIM), _full((FDIM, FDIM)), _full((FDIM, FDIM)), _full((1, FDIM))],
    out_specs=[_rows(FDIM), _rows(FDIM)],
    out_shape=[jax.ShapeDtypeStruct((N_NODES, FDIM), jnp.float32)] * 2,
)

_layer2 = pl.pallas_call(
    _layer2_body,
    grid=(GRID,),
    in_specs=[_rows(FDIM), _rows(FDIM), _rows(FDIM), _rows(1), _rows(1),
              _full((FDIM, FDIM)), _full((FDIM, FDIM)), _full((1, FDIM))],
    out_specs=[_rows(FDIM), _rows(FDIM), _rows(1)],
    out_shape=[jax.ShapeDtypeStruct((N_NODES, FDIM), jnp.float32),
               jax.ShapeDtypeStruct((N_NODES, FDIM), jnp.float32),
               jax.ShapeDtypeStruct((N_NODES, 1), jnp.float32)],
)

_layer3 = pl.pallas_call(
    _layer3_body,
    grid=(GRID,),
    in_specs=[_rows(FDIM), _rows(FDIM), _rows(FDIM), _rows(1),
              _full((FDIM, 1)), _full((FDIM, 1)), _full((1, 1))],
    out_specs=[_rows(1), _rows(1)],
    out_shape=[jax.ShapeDtypeStruct((N_NODES, 1), jnp.float32)] * 2,
)

_final = pl.pallas_call(
    _final_body,
    grid=(GRID,),
    in_specs=[_rows(1), _rows(1), _rows(1), _rows(1)],
    out_specs=_rows(1),
    out_shape=jax.ShapeDtypeStruct((N_NODES, 1), jnp.float32),
)


def kernel(x, edge_index, W1_self, W1_neigh, b1, W2_self, W2_neigh, b2,
           W3_self, W3_neigh, b3):
    src = edge_index[0].astype(jnp.int32)
    dst = edge_index[1].astype(jnp.int32)
    e = src.shape[0]
    srcp = jnp.concatenate([src, jnp.zeros((EPAD - e,), jnp.int32)]).reshape(TOT_CHUNKS, CH)
    dstp = jnp.concatenate([dst, jnp.full((EPAD - e,), NPAD - 1, jnp.int32)]).reshape(TOT_CHUNKS, CH)

    degp = _seg_scalar(jnp.ones((N_NODES,), jnp.float32), srcp, dstp)
    d0 = degp[0, :N_NODES, None]
    d1 = degp[1, :N_NODES, None]
    s1, p1 = _layer1(x, W1_self, W1_neigh, b1.reshape(1, FDIM))
    agg1 = _seg_rows(p1, srcp, dstp)
    s2, p2, invd = _layer2(s1, agg1[0, :N_NODES], agg1[1, :N_NODES], d0, d1,
                           W2_self, W2_neigh, b2.reshape(1, FDIM))
    agg2 = _seg_rows(p2, srcp, dstp)
    q3, p3 = _layer3(s2, agg2[0, :N_NODES], agg2[1, :N_NODES], invd,
                     W3_self, W3_neigh, b3.reshape(1, 1))
    agg3 = _seg_scalar(p3[:, 0], srcp, dstp)
    return _final(q3, agg3[0, :N_NODES, None], agg3[1, :N_NODES, None], invd)


# speedup vs baseline: 1.2018x; 1.0109x over previous
import functools

import jax
import jax.numpy as jnp
from jax import lax
from jax.experimental import pallas as pl
from jax.experimental.pallas import tpu as pltpu
from jax.experimental.pallas import tpu_sc as plsc

N_NODES = 10000
FDIM = 128
BLK = 1000
GRID = N_NODES // BLK

NCORES = 2
NSUB = 16
NWORK = NCORES * NSUB
CH = 128
CPW = 80
TOT_CHUNKS = NWORK * CPW
EPAD = TOT_CHUNKS * CH
NPAD = 10240
RPT = NPAD // NSUB
GC = 8
C0 = 152
C1 = 8

_mesh = plsc.VectorSubcoreMesh(core_axis_name="c", subcore_axis_name="s")


def _seg_rows_body(p_hbm, src_hbm, dst_hbm, out_hbm,
                   src_v, dst_v, rows_a, rows_b, zbuf, acc_sh, sem_a, sem_b):
    c = lax.axis_index("c")
    s = lax.axis_index("s")
    cnt = C0 + c * (C1 - C0)
    start = c * (NSUB * C0) + s * cnt
    ngroups = (C0 // GC) + c * ((C1 // GC) - (C0 // GC))

    zero16 = jnp.zeros((16,), jnp.float32)

    def _zrow(i, _):
        for j in range(FDIM // 16):
            zbuf[i, pl.ds(j * 16, 16)] = zero16
        return 0
    lax.fori_loop(0, 16, _zrow, 0)

    def _zacc(k, _):
        pltpu.sync_copy(zbuf, acc_sh.at[pl.ds(s * RPT + k * 16, 16)])
        return 0
    lax.fori_loop(0, RPT // 16, _zacc, 0)

    plsc.subcore_barrier()

    def _group(g, _):
        pltpu.sync_copy(src_hbm.at[pl.ds(start + g * GC, GC)], src_v)
        pltpu.sync_copy(dst_hbm.at[pl.ds(start + g * GC, GC)], dst_v)
        copies = [None, None]
        bufs = [rows_a, rows_b]
        copies[0] = pltpu.async_copy(p_hbm.at[src_v.at[0]], rows_a, sem_a)
        for j in range(GC):
            if j + 1 < GC:
                copies[(j + 1) % 2] = pltpu.async_copy(
                    p_hbm.at[src_v.at[j + 1]], bufs[(j + 1) % 2], sem_b)
            copies[j % 2].wait()
            pltpu.sync_copy(bufs[j % 2], acc_sh.at[dst_v.at[j]], add=True)
        return 0
    lax.fori_loop(0, ngroups, _group, 0)

    plsc.subcore_barrier()

    pltpu.sync_copy(acc_sh.at[pl.ds(s * RPT, RPT)],
                    out_hbm.at[c, pl.ds(s * RPT, RPT)])


_seg_rows = functools.partial(
    pl.kernel, _seg_rows_body, mesh=_mesh,
    out_type=jax.ShapeDtypeStruct((NCORES, NPAD, FDIM), jnp.float32),
    scratch_types=[
        pltpu.VMEM((GC, CH), jnp.int32),
        pltpu.VMEM((GC, CH), jnp.int32),
        pltpu.VMEM((CH, FDIM), jnp.float32),
        pltpu.VMEM((CH, FDIM), jnp.float32),
        pltpu.VMEM((16, FDIM), jnp.float32),
        pltpu.VMEM_SHARED((NPAD, FDIM), jnp.float32),
        pltpu.SemaphoreType.DMA,
        pltpu.SemaphoreType.DMA,
    ],
)()


def _seg_scalar_body(p_hbm, src_hbm, dst_hbm, out_hbm,
                     src_v, dst_v, vals_v, vals_w, z1d, acc_sh, sem, sem2):
    c = lax.axis_index("c")
    s = lax.axis_index("s")
    wid = c * NSUB + s

    zero16 = jnp.zeros((16,), jnp.float32)

    def _z1(i, _):
        z1d[pl.ds(i * 16, 16)] = zero16
        return 0
    lax.fori_loop(0, RPT // 16, _z1, 0)
    pltpu.sync_copy(z1d, acc_sh.at[pl.ds(s * RPT, RPT)])

    pltpu.sync_copy(src_hbm.at[pl.ds(wid * CPW, CPW)], src_v)
    pltpu.sync_copy(dst_hbm.at[pl.ds(wid * CPW, CPW)], dst_v)

    plsc.subcore_barrier()

    def _vgroup(g, _):
        base = g * GC
        copies = [None, None]
        bufs = [vals_v, vals_w]
        copies[0] = pltpu.async_copy(p_hbm.at[src_v.at[base]], vals_v, sem)
        for j in range(GC):
            if j + 1 < GC:
                copies[(j + 1) % 2] = pltpu.async_copy(
                    p_hbm.at[src_v.at[base + j + 1]], bufs[(j + 1) % 2], sem2)
            copies[j % 2].wait()
            pltpu.sync_copy(bufs[j % 2], acc_sh.at[dst_v.at[base + j]], add=True)
        return 0
    lax.fori_loop(0, CPW // GC, _vgroup, 0)

    plsc.subcore_barrier()

    pltpu.sync_copy(acc_sh.at[pl.ds(s * RPT, RPT)],
                    out_hbm.at[c, pl.ds(s * RPT, RPT)])


_seg_scalar = functools.partial(
    pl.kernel, _seg_scalar_body, mesh=_mesh,
    out_type=jax.ShapeDtypeStruct((NCORES, NPAD), jnp.float32),
    scratch_types=[
        pltpu.VMEM((CPW, CH), jnp.int32),
        pltpu.VMEM((CPW, CH), jnp.int32),
        pltpu.VMEM((CH,), jnp.float32),
        pltpu.VMEM((CH,), jnp.float32),
        pltpu.VMEM((RPT,), jnp.float32),
        pltpu.VMEM_SHARED((NPAD,), jnp.float32),
        pltpu.SemaphoreType.DMA,
        pltpu.SemaphoreType.DMA,
    ],
)()


def _layer1_body(x_ref, ws_ref, wn_ref, b_ref, s1_ref, p1_ref):
    xb = x_ref[...]
    s1_ref[...] = jnp.dot(xb, ws_ref[...], preferred_element_type=jnp.float32) + b_ref[...]
    p1_ref[...] = jnp.dot(xb, wn_ref[...], preferred_element_type=jnp.float32)


def _layer2_body(s1_ref, a0_ref, a1_ref, d0_ref, d1_ref, ws_ref, wn_ref, b_ref,
                 s2_ref, p2_ref, invd_ref):
    deg = d0_ref[...] + d1_ref[...]
    inv = 1.0 / jnp.maximum(deg, 1.0)
    h1 = jnp.maximum(s1_ref[...] + (a0_ref[...] + a1_ref[...]) * inv, 0.0)
    s2_ref[...] = jnp.dot(h1, ws_ref[...], preferred_element_type=jnp.float32) + b_ref[...]
    p2_ref[...] = jnp.dot(h1, wn_ref[...], preferred_element_type=jnp.float32)
    invd_ref[...] = inv


def _layer3_body(s2_ref, a0_ref, a1_ref, invd_ref, ws_ref, wn_ref, b_ref,
                 q3_ref, p3_ref):
    inv = invd_ref[...]
    h2 = jnp.maximum(s2_ref[...] + (a0_ref[...] + a1_ref[...]) * inv, 0.0)
    q3_ref[...] = jnp.dot(h2, ws_ref[...], preferred_element_type=jnp.float32) + b_ref[...]
    p3_ref[...] = jnp.dot(h2, wn_ref[...], preferred_element_type=jnp.float32)


def _final_body(q3_ref, a0_ref, a1_ref, invd_ref, out_ref):
    out_ref[...] = q3_ref[...] + (a0_ref[...] + a1_ref[...]) * invd_ref[...]


def _full(shape):
    return pl.BlockSpec(shape, lambda i: tuple(0 for _ in shape))


def _rows(width):
    return pl.BlockSpec((BLK, width), lambda i: (i, 0))


_layer1 = pl.pallas_call(
    _layer1_body,
    grid=(GRID,),
    in_specs=[_rows(FDIM), _full((FDIM, FDIM)), _full((FDIM, FDIM)), _full((1, FDIM))],
    out_specs=[_rows(FDIM), _rows(FDIM)],
    out_shape=[jax.ShapeDtypeStruct((N_NODES, FDIM), jnp.float32)] * 2,
)

_layer2 = pl.pallas_call(
    _layer2_body,
    grid=(GRID,),
    in_specs=[_rows(FDIM), _rows(FDIM), _rows(FDIM), _rows(1), _rows(1),
              _full((FDIM, FDIM)), _full((FDIM, FDIM)), _full((1, FDIM))],
    out_specs=[_rows(FDIM), _rows(FDIM), _rows(1)],
    out_shape=[jax.ShapeDtypeStruct((N_NODES, FDIM), jnp.float32),
               jax.ShapeDtypeStruct((N_NODES, FDIM), jnp.float32),
               jax.ShapeDtypeStruct((N_NODES, 1), jnp.float32)],
)

_layer3 = pl.pallas_call(
    _layer3_body,
    grid=(GRID,),
    in_specs=[_rows(FDIM), _rows(FDIM), _rows(FDIM), _rows(1),
              _full((FDIM, 1)), _full((FDIM, 1)), _full((1, 1))],
    out_specs=[_rows(1), _rows(1)],
    out_shape=[jax.ShapeDtypeStruct((N_NODES, 1), jnp.float32)] * 2,
)

_final = pl.pallas_call(
    _final_body,
    grid=(GRID,),
    in_specs=[_rows(1), _rows(1), _rows(1), _rows(1)],
    out_specs=_rows(1),
    out_shape=jax.ShapeDtypeStruct((N_NODES, 1), jnp.float32),
)


def kernel(x, edge_index, W1_self, W1_neigh, b1, W2_self, W2_neigh, b2,
           W3_self, W3_neigh, b3):
    src = edge_index[0].astype(jnp.int32)
    dst = edge_index[1].astype(jnp.int32)
    e = src.shape[0]
    srcp = jnp.concatenate([src, jnp.zeros((EPAD - e,), jnp.int32)]).reshape(TOT_CHUNKS, CH)
    dstp = jnp.concatenate([dst, jnp.full((EPAD - e,), NPAD - 1, jnp.int32)]).reshape(TOT_CHUNKS, CH)

    degp = _seg_scalar(jnp.ones((N_NODES,), jnp.float32), srcp, dstp)
    d0 = degp[0, :N_NODES, None]
    d1 = degp[1, :N_NODES, None]
    s1, p1 = _layer1(x, W1_self, W1_neigh, b1.reshape(1, FDIM))
    agg1 = _seg_rows(p1, srcp, dstp)
    s2, p2, invd = _layer2(s1, agg1[0, :N_NODES], agg1[1, :N_NODES], d0, d1,
                           W2_self, W2_neigh, b2.reshape(1, FDIM))
    agg2 = _seg_rows(p2, srcp, dstp)
    q3, p3 = _layer3(s2, agg2[0, :N_NODES], agg2[1, :N_NODES], invd,
                     W3_self, W3_neigh, b3.reshape(1, 1))
    agg3 = _seg_scalar(p3[:, 0], srcp, dstp)
    return _final(q3, agg3[0, :N_NODES, None], agg3[1, :N_NODES, None], invd)
